# Initial kernel scaffold; baseline (speedup 1.0000x reference)
import jax
import jax.numpy as jnp
from jax.experimental import pallas as pl


def kernel(features, weights, shared_table, imp_table):
    raise NotImplementedError("write your pallas kernel here")


# speedup vs baseline: 10.9285x; 10.9285x over previous
import functools

import jax
import jax.numpy as jnp
import numpy as np
from jax import lax
from jax.experimental import pallas as pl
from jax.experimental.pallas import tpu as pltpu
from jax.experimental.pallas import tpu_sc as plsc

_VOCAB = 1000000
_NUM_EMB = _VOCAB + 1
_D = 64
_H = 2
_BUCKETS = 100003
_MODULER = 2147483647
_HASH_A = (1103515245, 214013)
_HASH_B = (12345, 2531011)
_z = np.int32(0)


def _sc_gather(table_view, idx_flat, n_total):
    info = plsc.get_sparse_core_info()
    nw = info.num_cores * info.num_subcores
    n_per_w = n_total // nw
    chunk = 800
    steps = n_per_w // chunk
    mesh = plsc.VectorSubcoreMesh(core_axis_name="c", subcore_axis_name="s")

    @functools.partial(
        pl.kernel,
        mesh=mesh,
        out_type=jax.ShapeDtypeStruct((n_total, 2 * _D), jnp.float32),
        scratch_types=[
            pltpu.VMEM((chunk,), jnp.int32),
            pltpu.VMEM((chunk, 2 * _D), jnp.float32),
            pltpu.SemaphoreType.DMA,
        ],
    )
    def gather_kernel(table_hbm, idx_hbm, out_hbm, idx_v, rows_v, sem):
        wid = lax.axis_index("s") * jnp.int32(info.num_cores) + lax.axis_index("c")
        base = wid * jnp.int32(n_per_w)

        def body(i, carry):
            off = base + i * chunk
            pltpu.sync_copy(idx_hbm.at[pl.ds(off, chunk)], idx_v)
            pltpu.async_copy(table_hbm.at[idx_v], rows_v, sem).wait()
            pltpu.sync_copy(rows_v, out_hbm.at[pl.ds(off, chunk)])
            return carry

        lax.fori_loop(jnp.int32(0), jnp.int32(steps), body, jnp.int32(0))

    return gather_kernel(table_view, idx_flat)


def _tc_reduce(gathered, clo, chi, b, l):
    blk = 256
    jdim = l * _H

    def reduce_kernel(g_ref, clo_ref, chi_ref, o_ref):
        g = g_ref[...]
        cl = clo_ref[...]
        ch = chi_ref[...]
        main = jnp.sum(
            g[:, :, :_D] * cl[:, :, None] + g[:, :, _D:] * ch[:, :, None], axis=1
        )
        csum = cl + ch
        j_iota = lax.broadcasted_iota(jnp.int32, (blk, jdim), 1)
        even = (j_iota % 2) == 0
        t0 = jnp.sum(jnp.where(even, csum, 0.0), axis=1)
        t1 = jnp.sum(jnp.where(even, 0.0, csum), axis=1)
        o_ref[...] = jnp.concatenate([main, t0[:, None], t1[:, None]], axis=-1)

    return pl.pallas_call(
        reduce_kernel,
        grid=(b // blk,),
        in_specs=[
            pl.BlockSpec((blk, jdim, 2 * _D), lambda i: (i, _z, _z)),
            pl.BlockSpec((blk, jdim), lambda i: (i, _z)),
            pl.BlockSpec((blk, jdim), lambda i: (i, _z)),
        ],
        out_specs=pl.BlockSpec((blk, _D + _H), lambda i: (i, _z)),
        out_shape=jax.ShapeDtypeStruct((b, _D + _H), jnp.float32),
    )(gathered, clo, chi)


def kernel(features, weights, shared_table, imp_table):
    b, l = features.shape
    f64 = features.astype(jnp.int64)
    a = jnp.asarray(_HASH_A, dtype=jnp.int64)
    bb = jnp.asarray(_HASH_B, dtype=jnp.int64)
    hashed = (a[None, None, :] * f64[:, :, None] + bb[None, None, :]) % _MODULER % _BUCKETS
    idx_shared = jnp.where(f64[:, :, None] == 0, 0, hashed).astype(jnp.int32)
    idx_flat = idx_shared.reshape(b * l * _H)
    pair_idx = idx_flat >> 1
    parity = (idx_flat & 1).astype(jnp.float32).reshape(b, l * _H)

    idx_imp = (f64 % _NUM_EMB).astype(jnp.int32)
    importance = jax.nn.sigmoid(imp_table[idx_imp])
    coeffs = (weights[:, :, None] * importance).reshape(b, l * _H)
    clo = coeffs * (1.0 - parity)
    chi = coeffs * parity

    table_view = shared_table.reshape(-1, 2 * _D)
    gathered = _sc_gather(table_view, pair_idx, b * l * _H)
    gathered = gathered.reshape(b, l * _H, 2 * _D)
    return _tc_reduce(gathered, clo, chi, b, l)
